# baseline (speedup 1.0000x reference)
import functools
import jax
import jax.numpy as jnp
from jax import lax
from jax.experimental import pallas as pl
from jax.experimental.pallas import tpu as pltpu
from jax.experimental.pallas import tpu_sc as plsc

B, N, F = 4, 8192, 2048
NW = 32
WPB = NW // B
FW = F // WPB
R = 128
NT = N // R
NV = FW // 16

_mesh = plsc.VectorSubcoreMesh(core_axis_name="c", subcore_axis_name="s")


@functools.partial(
    pl.kernel,
    mesh=_mesh,
    out_type=jax.ShapeDtypeStruct((B, N, F), jnp.float32),
    scratch_types=[
        pltpu.VMEM((2, R, FW), jnp.float32),
        pltpu.SemaphoreType.DMA,
        pltpu.SemaphoreType.DMA,
        pltpu.SemaphoreType.DMA,
        pltpu.SemaphoreType.DMA,
    ],
)
def _sc_cumsum(x_hbm, out_hbm, buf, lsem0, lsem1, ssem0, ssem1):
    wid = lax.axis_index("s") * 2 + lax.axis_index("c")
    b = wid // WPB
    f0 = (wid % WPB) * FW
    lsems = [lsem0, lsem1]
    ssems = [ssem0, ssem1]

    def load_copy(t, s):
        return pltpu.make_async_copy(
            x_hbm.at[b, pl.ds(t * R, R), pl.ds(f0, FW)],
            buf.at[s],
            lsems[s],
        )

    def store_copy(t, s):
        return pltpu.make_async_copy(
            buf.at[s],
            out_hbm.at[b, pl.ds(t * R, R), pl.ds(f0, FW)],
            ssems[s],
        )

    load_copy(0, 0).start()
    load_copy(1, 1).start()

    def phase(t, s, carry):
        load_copy(t, s).wait()

        def row(r, acc):
            new = []
            for j in range(NV):
                v = acc[j] + buf[s, r, pl.ds(16 * j, 16)]
                buf[s, r, pl.ds(16 * j, 16)] = v
                new.append(v)
            return tuple(new)

        carry = lax.fori_loop(0, R, row, carry, unroll=2)
        store_copy(t, s).start()
        store_copy(t, s).wait()

        @pl.when(t + 2 < NT)
        def _():
            load_copy(t + 2, s).start()

        return carry

    def two(i, carry):
        t = i * 2
        carry = phase(t, 0, carry)
        carry = phase(t + 1, 1, carry)
        return carry

    zeros = tuple(jnp.zeros((16,), jnp.float32) for _ in range(NV))
    lax.fori_loop(0, NT // 2, two, zeros)


def kernel(x):
    return _sc_cumsum(x)
